# baseline (speedup 1.0000x reference)
import jax
import jax.numpy as jnp
import numpy as np
from jax.experimental import pallas as pl
from jax.experimental.pallas import tpu as pltpu

TOKENS = 8192
D_MODEL = 4096
D_HID = 4096
N_EXPERTS = 64

TOK_BLK = 512
HID_BLK = 512

_SQRT_HALF = np.sqrt(0.5).astype(np.float32)

_ERF_T = [7.853861353153693e-5, -8.010193625184903e-4, 5.188327685732524e-3,
          -2.685381193529856e-2, 1.128358514861418e-1, -3.761262582423300e-1,
          1.128379165726710e+0]
_ERFC_P = [2.326819970068386e-2, -1.387039388740657e-1, 3.687424674597105e-1,
           -5.824733027278666e-1, 6.210004621745983e-1, -4.944515323274145e-1,
           3.404879937665872e-1, -2.741127028184656e-1, 5.638259427386472e-1]
_ERFC_R = [-1.047766399936249e+1, 1.297719955372516e+1, -7.495518717768503e+0,
           2.921019019210786e+0, -1.015265279202700e+0, 4.218463358204948e-1,
           -2.820767439740514e-1, 5.641895067754075e-1]


def _poly(x, coeffs):
    p = jnp.full_like(x, np.float32(coeffs[0]))
    for c in coeffs[1:]:
        p = p * x + np.float32(c)
    return p


def _erfc(x):
    ax = jnp.abs(x)
    x2 = x * x
    z = jnp.exp(-x2)
    q = 1.0 / ax
    y2 = 1.0 / x2
    p = jnp.where(ax < 2.0, _poly(y2, _ERFC_P), _poly(y2, _ERFC_R))
    y = (z * q) * p
    big = jnp.where(x < 0.0, 2.0 - y, y)
    small = 1.0 - x * _poly(x2, _ERF_T)
    return jnp.where(ax < 1.0, small, big)


def _gelu(v):
    return 0.5 * v * _erfc(-v * _SQRT_HALF)


def _router_kernel(x_ref, w1_ref, b1_ref, w2_ref, b2_ref,
                   wout_ref, iout_ref, xb_ref, hb_ref):
    j = pl.program_id(1)
    nj = pl.num_programs(1)

    @pl.when(j == 0)
    def _pack_x():
        xb_ref[...] = x_ref[...].astype(jnp.bfloat16)

    xb = xb_ref[...]
    w1b = w1_ref[...].astype(jnp.bfloat16)
    h = jax.lax.dot_general(
        xb, w1b, (((1,), (1,)), ((), ())),
        preferred_element_type=jnp.float32)
    h = _gelu(h + b1_ref[...])
    hb_ref[:, pl.ds(j * HID_BLK, HID_BLK)] = h.astype(jnp.bfloat16)

    @pl.when(j == nj - 1)
    def _epilogue():
        acc = jax.lax.dot_general(
            hb_ref[...], w2_ref[...].astype(jnp.bfloat16),
            (((1,), (1,)), ((), ())),
            preferred_element_type=jnp.float32)
        logits = _gelu(acc + b2_ref[...])
        idx = jax.lax.broadcasted_iota(jnp.int32, logits.shape, 1)
        m1 = jnp.max(logits, axis=1, keepdims=True)
        i1 = jnp.min(jnp.where(logits == m1, idx, N_EXPERTS),
                     axis=1, keepdims=True)
        masked = jnp.where(idx == i1, -jnp.inf, logits)
        m2 = jnp.max(masked, axis=1, keepdims=True)
        i2 = jnp.min(jnp.where(masked == m2, idx, N_EXPERTS),
                     axis=1, keepdims=True)
        e2 = jnp.exp(m2 - m1)
        denom = 1.0 + e2
        wout_ref[...] = jnp.concatenate([1.0 / denom, e2 / denom], axis=1)
        iout_ref[...] = jnp.concatenate([i1, i2], axis=1)


@jax.jit
def kernel(x, W1, b1, W2, b2):
    n_tok = TOKENS // TOK_BLK
    n_hid = D_HID // HID_BLK
    b1r = b1.reshape(1, D_HID)
    b2r = b2.reshape(1, N_EXPERTS)
    grid = (n_tok, n_hid)
    weights, indexes = pl.pallas_call(
        _router_kernel,
        grid=grid,
        in_specs=[
            pl.BlockSpec((TOK_BLK, D_MODEL), lambda i, j: (i, 0)),
            pl.BlockSpec((HID_BLK, D_MODEL), lambda i, j: (j, 0)),
            pl.BlockSpec((1, HID_BLK), lambda i, j: (0, j)),
            pl.BlockSpec((N_EXPERTS, D_HID), lambda i, j: (0, 0)),
            pl.BlockSpec((1, N_EXPERTS), lambda i, j: (0, 0)),
        ],
        out_specs=[
            pl.BlockSpec((TOK_BLK, 2), lambda i, j: (i, 0)),
            pl.BlockSpec((TOK_BLK, 2), lambda i, j: (i, 0)),
        ],
        out_shape=[
            jax.ShapeDtypeStruct((TOKENS, 2), jnp.float32),
            jax.ShapeDtypeStruct((TOKENS, 2), jnp.int32),
        ],
        scratch_shapes=[pltpu.VMEM((TOK_BLK, D_MODEL), jnp.bfloat16),
                        pltpu.VMEM((TOK_BLK, D_HID), jnp.bfloat16)],
        compiler_params=pltpu.CompilerParams(
            dimension_semantics=("parallel", "arbitrary"),
        ),
    )(x, W1, b1r, W2, b2r)
    return (weights, indexes)
